# baseline (speedup 1.0000x reference)
import jax
import jax.numpy as jnp
from jax import lax
from jax.experimental import pallas as pl
from jax.experimental.pallas import tpu as pltpu
from jax.experimental.pallas import tpu_sc as plsc

NC = 2
NS = 16
LANES = 16
NW = NC * NS

B = 16384
L = 20
D = 64
CHUNK = 128
ROWS_W = B // NW
NCH_ARR = (ROWS_W * L) // CHUNK
NCH = 3 * NCH_ARR
NBUF = 3
NGRP = NCH // NBUF
REGION = 520
TRASH = 512


def _body(sidx, sw, vidx, vw, oidx, ow, table, out,
          ids_v, w_v, dst_v, acc_ref, bufs, gsems, ssems):
    obuf = bufs[0]
    c = lax.axis_index("c")
    s = lax.axis_index("s")
    w = s * NC + c
    sbase = s * REGION

    pltpu.sync_copy(sidx.at[w], ids_v.at[pl.ds(0 * NCH_ARR, NCH_ARR)])
    pltpu.sync_copy(vidx.at[w], ids_v.at[pl.ds(1 * NCH_ARR, NCH_ARR)])
    pltpu.sync_copy(oidx.at[w], ids_v.at[pl.ds(2 * NCH_ARR, NCH_ARR)])

    for b in range(1, NBUF):
        pltpu.async_copy(table.at[ids_v.at[b]], bufs[b], gsems[b])

    zero = jnp.zeros((LANES,), jnp.float32)

    def zrow(r, carry):
        for q in range(D // LANES):
            obuf[r, pl.ds(q * LANES, LANES)] = zero
        return carry

    lax.fori_loop(0, CHUNK, zrow, 0)
    for t in range(4):
        pltpu.sync_copy(obuf, acc_ref.at[pl.ds(sbase + t * CHUNK, CHUNK)])
    pltpu.sync_copy(obuf.at[pl.ds(0, 8)],
                    acc_ref.at[pl.ds(sbase + 4 * CHUNK, 8)])
    pltpu.async_copy(table.at[ids_v.at[0]], bufs[0], gsems[0])

    iota = lax.iota(jnp.int32, LANES)
    trash_vec = jnp.full((LANES,), sbase + TRASH, jnp.int32)
    for a, w_hbm in enumerate((sw, vw, ow)):
        pltpu.sync_copy(w_hbm.at[w], w_v)

        def dst_body(i, carry, _a=a):
            for q in range(CHUNK // LANES):
                ent = iota + (i * CHUNK + q * LANES)
                row = lax.div(ent, jnp.int32(L))
                wv = w_v[i, pl.ds(q * LANES, LANES)]
                dst = jnp.where(wv != 0.0, sbase + row, trash_vec)
                dst_v[_a * NCH_ARR + i, pl.ds(q * LANES, LANES)] = dst
            return carry

        lax.fori_loop(0, NCH_ARR, dst_body, 0)

    def grp_body(g, carry):
        for b in range(NBUF):
            i = g * NBUF + b
            pltpu.make_async_copy(table.at[ids_v.at[i]], bufs[b],
                                  gsems[b]).wait()
            pltpu.async_copy(bufs[b], acc_ref.at[dst_v.at[i]], ssems[b],
                             add=True)

        @pl.when(g < NGRP - 1)
        def _refill():
            for b in range(NBUF):
                i = (g + 1) * NBUF + b
                pltpu.make_async_copy(bufs[b], acc_ref.at[dst_v.at[i]],
                                      ssems[b]).wait()
                pltpu.async_copy(table.at[ids_v.at[i]], bufs[b], gsems[b])

        return carry

    lax.fori_loop(0, NGRP, grp_body, 0)

    for b in range(NBUF):
        i = NCH - NBUF + b
        pltpu.make_async_copy(bufs[b], acc_ref.at[dst_v.at[i]],
                              ssems[b]).wait()

    for t in range(4):
        pltpu.sync_copy(acc_ref.at[pl.ds(sbase + t * CHUNK, CHUNK)], obuf)
        pltpu.sync_copy(obuf, out.at[pl.ds(w * ROWS_W + t * CHUNK, CHUNK)])


_A_BR = 2000


def _repack_block(in_ref, out_ref):
    x = in_ref[...].reshape(_A_BR // 2, 2, D)
    out_ref[:, 0:D] = x[:, 0, :]
    out_ref[:, D:2 * D] = x[:, 1, :]


def _repack_tc(table):
    return pl.pallas_call(
        _repack_block,
        out_shape=jax.ShapeDtypeStruct((500000, 2 * D), jnp.float32),
        grid=(1000000 // _A_BR,),
        in_specs=[pl.BlockSpec((_A_BR, D), lambda i: (i, 0))],
        out_specs=pl.BlockSpec((_A_BR // 2, 2 * D), lambda i: (i, 0)),
    )(table)


@jax.jit
def _run(sidx, sw, vidx, vw, oidx, ow, table):
    table = jnp.reshape(_repack_tc(table), (1000000, D))
    mesh = plsc.VectorSubcoreMesh(core_axis_name="c", subcore_axis_name="s")

    def body(sidx, sw, vidx, vw, oidx, ow, table, out,
             ids_v, w_v, dst_v, acc,
             b0, b1, b2, g0, g1, g2, s0, s1, s2):
        _body(sidx, sw, vidx, vw, oidx, ow, table, out,
              ids_v, w_v, dst_v, acc,
              (b0, b1, b2), (g0, g1, g2), (s0, s1, s2))

    f = pl.kernel(
        body,
        out_type=jax.ShapeDtypeStruct((B, D), jnp.float32),
        mesh=mesh,
        scratch_types=[
            pltpu.VMEM((NCH, CHUNK), jnp.int32),
            pltpu.VMEM((NCH_ARR, CHUNK), jnp.float32),
            pltpu.VMEM((NCH, CHUNK), jnp.int32),
            pltpu.VMEM_SHARED((NS * REGION, D), jnp.float32),
        ] + [pltpu.VMEM((CHUNK, D), jnp.float32) for _ in range(NBUF)]
          + [pltpu.SemaphoreType.DMA for _ in range(2 * NBUF)],
        compiler_params=pltpu.CompilerParams(use_tc_tiling_on_sc=False),
    )
    return f(sidx, sw, vidx, vw, oidx, ow, table)


def kernel(subj_id, subj_w, verb_id, verb_w, obj_id, obj_w, table):
    shp = (NW, NCH_ARR, CHUNK)
    return _run(
        subj_id.astype(jnp.int32).reshape(shp), subj_w.reshape(shp),
        verb_id.astype(jnp.int32).reshape(shp), verb_w.reshape(shp),
        obj_id.astype(jnp.int32).reshape(shp), obj_w.reshape(shp),
        table)


# speedup vs baseline: 1.4637x; 1.4637x over previous
import jax
import jax.numpy as jnp
from jax import lax
from jax.experimental import pallas as pl
from jax.experimental.pallas import tpu as pltpu
from jax.experimental.pallas import tpu_sc as plsc

NC = 2
NS = 16
LANES = 16
NW = NC * NS

B = 16384
L = 20
D = 64
CHUNK = 128
ROWS_W = B // NW
NCH_ARR = (ROWS_W * L) // CHUNK
NCH = 3 * NCH_ARR
NBUF = 4
NGRP = NCH // NBUF
REGION = 520
TRASH = 512


def _body(sidx, sw, vidx, vw, oidx, ow, table, out,
          ids_v, dst_v, acc_ref, bufs, gsems, ssems):
    obuf = bufs[0]
    c = lax.axis_index("c")
    s = lax.axis_index("s")
    w = s * NC + c
    sbase = s * REGION

    pltpu.sync_copy(sidx.at[w], ids_v.at[pl.ds(0 * NCH_ARR, NCH_ARR)])
    pltpu.sync_copy(vidx.at[w], ids_v.at[pl.ds(1 * NCH_ARR, NCH_ARR)])
    pltpu.sync_copy(oidx.at[w], ids_v.at[pl.ds(2 * NCH_ARR, NCH_ARR)])
    pltpu.sync_copy(sw.at[w], dst_v.at[pl.ds(0 * NCH_ARR, NCH_ARR)])
    pltpu.sync_copy(vw.at[w], dst_v.at[pl.ds(1 * NCH_ARR, NCH_ARR)])
    pltpu.sync_copy(ow.at[w], dst_v.at[pl.ds(2 * NCH_ARR, NCH_ARR)])

    for b in range(1, NBUF):
        pltpu.async_copy(table.at[ids_v.at[b]], bufs[b], gsems[b])

    zero = jnp.zeros((LANES,), jnp.float32)

    def zrow(r, carry):
        for q in range(D // LANES):
            obuf[r, pl.ds(q * LANES, LANES)] = zero
        return carry

    lax.fori_loop(0, CHUNK, zrow, 0)
    for t in range(4):
        pltpu.sync_copy(obuf, acc_ref.at[pl.ds(sbase + t * CHUNK, CHUNK)])
    pltpu.sync_copy(obuf.at[pl.ds(0, 8)],
                    acc_ref.at[pl.ds(sbase + 4 * CHUNK, 8)])
    pltpu.async_copy(table.at[ids_v.at[0]], bufs[0], gsems[0])

    iota = lax.iota(jnp.int32, LANES)
    trash_vec = jnp.full((LANES,), sbase + TRASH, jnp.int32)

    def dst_body(i, carry):
        for q in range(CHUNK // LANES):
            ent = iota + (lax.rem(i, NCH_ARR) * CHUNK + q * LANES)
            row = lax.div(ent, jnp.int32(L))
            wbits = dst_v[i, pl.ds(q * LANES, LANES)]
            wv = plsc.bitcast(wbits, jnp.float32)
            dst = jnp.where(wv != 0.0, sbase + row, trash_vec)
            dst_v[i, pl.ds(q * LANES, LANES)] = dst
        return carry

    lax.fori_loop(0, NCH, dst_body, 0)

    def grp_body(g, carry):
        for b in range(NBUF):
            i = g * NBUF + b
            pltpu.make_async_copy(table.at[ids_v.at[i]], bufs[b],
                                  gsems[b]).wait()
            pltpu.async_copy(bufs[b], acc_ref.at[dst_v.at[i]], ssems[b],
                             add=True)

        @pl.when(g < NGRP - 1)
        def _refill():
            for b in range(NBUF):
                i = (g + 1) * NBUF + b
                pltpu.make_async_copy(bufs[b], acc_ref.at[dst_v.at[i]],
                                      ssems[b]).wait()
                pltpu.async_copy(table.at[ids_v.at[i]], bufs[b], gsems[b])

        return carry

    lax.fori_loop(0, NGRP, grp_body, 0)

    for b in range(NBUF):
        i = NCH - NBUF + b
        pltpu.make_async_copy(bufs[b], acc_ref.at[dst_v.at[i]],
                              ssems[b]).wait()

    for t in range(4):
        pltpu.sync_copy(acc_ref.at[pl.ds(sbase + t * CHUNK, CHUNK)], obuf)
        pltpu.sync_copy(obuf, out.at[pl.ds(w * ROWS_W + t * CHUNK, CHUNK)])


@jax.jit
def _run(sidx, sw, vidx, vw, oidx, ow, table):
    mesh = plsc.VectorSubcoreMesh(core_axis_name="c", subcore_axis_name="s")

    def body(sidx, sw, vidx, vw, oidx, ow, table, out,
             ids_v, dst_v, acc,
             b0, b1, b2, b3, g0, g1, g2, g3, s0, s1, s2, s3):
        _body(sidx, sw, vidx, vw, oidx, ow, table, out,
              ids_v, dst_v, acc,
              (b0, b1, b2, b3), (g0, g1, g2, g3), (s0, s1, s2, s3))

    f = pl.kernel(
        body,
        out_type=jax.ShapeDtypeStruct((B, D), jnp.float32),
        mesh=mesh,
        scratch_types=[
            pltpu.VMEM((NCH, CHUNK), jnp.int32),
            pltpu.VMEM((NCH, CHUNK), jnp.int32),
            pltpu.VMEM_SHARED((NS * REGION, D), jnp.float32),
        ] + [pltpu.VMEM((CHUNK, D), jnp.float32) for _ in range(NBUF)]
          + [pltpu.SemaphoreType.DMA for _ in range(2 * NBUF)],
        compiler_params=pltpu.CompilerParams(use_tc_tiling_on_sc=False,
                                             needs_layout_passes=False),
    )
    return f(sidx, sw, vidx, vw, oidx, ow, table)


def kernel(subj_id, subj_w, verb_id, verb_w, obj_id, obj_w, table):
    shp = (NW, NCH_ARR, CHUNK)

    def wb(x):
        return lax.bitcast_convert_type(x, jnp.int32).reshape(shp)

    return _run(
        subj_id.astype(jnp.int32).reshape(shp), wb(subj_w),
        verb_id.astype(jnp.int32).reshape(shp), wb(verb_w),
        obj_id.astype(jnp.int32).reshape(shp), wb(obj_w),
        table)


# speedup vs baseline: 1.4668x; 1.0021x over previous
import jax
import jax.numpy as jnp
from jax import lax
from jax.experimental import pallas as pl
from jax.experimental.pallas import tpu as pltpu
from jax.experimental.pallas import tpu_sc as plsc

NC = 2
NS = 16
LANES = 16
NW = NC * NS

B = 16384
L = 20
D = 64
CHUNK = 128
ROWS_W = B // NW
NCH_ARR = (ROWS_W * L) // CHUNK
NCH = 3 * NCH_ARR
NBUF = 4
NGRP = NCH // NBUF
REGION = 520
TRASH = 512


def _body(sidx, sw, vidx, vw, oidx, ow, table, out,
          ids_v, dst_v, acc_ref, bufs, gsems, ssems):
    obuf = bufs[0]
    c = lax.axis_index("c")
    s = lax.axis_index("s")
    w = s * NC + c
    sbase = s * REGION

    pltpu.sync_copy(sidx.at[w], ids_v.at[pl.ds(0 * NCH_ARR, NCH_ARR)])
    pltpu.sync_copy(vidx.at[w], ids_v.at[pl.ds(1 * NCH_ARR, NCH_ARR)])
    pltpu.sync_copy(oidx.at[w], ids_v.at[pl.ds(2 * NCH_ARR, NCH_ARR)])
    pltpu.sync_copy(sw.at[w], dst_v.at[pl.ds(0 * NCH_ARR, NCH_ARR)])
    pltpu.sync_copy(vw.at[w], dst_v.at[pl.ds(1 * NCH_ARR, NCH_ARR)])
    pltpu.sync_copy(ow.at[w], dst_v.at[pl.ds(2 * NCH_ARR, NCH_ARR)])

    for b in range(1, NBUF):
        pltpu.async_copy(table.at[ids_v.at[b]], bufs[b], gsems[b])

    zero = jnp.zeros((LANES,), jnp.float32)

    def zrow(r, carry):
        for q in range(D // LANES):
            obuf[r, pl.ds(q * LANES, LANES)] = zero
        return carry

    lax.fori_loop(0, CHUNK, zrow, 0)
    for t in range(4):
        pltpu.sync_copy(obuf, acc_ref.at[pl.ds(sbase + t * CHUNK, CHUNK)])
    pltpu.sync_copy(obuf.at[pl.ds(0, 8)],
                    acc_ref.at[pl.ds(sbase + 4 * CHUNK, 8)])
    pltpu.async_copy(table.at[ids_v.at[0]], bufs[0], gsems[0])

    iota = lax.iota(jnp.int32, LANES)
    trash_vec = jnp.full((LANES,), sbase + TRASH, jnp.int32)

    def compute_dst(i):
        for q in range(CHUNK // LANES):
            ent = iota + (lax.rem(i, NCH_ARR) * CHUNK + q * LANES)
            row = lax.div(ent, jnp.int32(L))
            wbits = dst_v[i, pl.ds(q * LANES, LANES)]
            wv = plsc.bitcast(wbits, jnp.float32)
            dst = jnp.where(wv != 0.0, sbase + row, trash_vec)
            dst_v[i, pl.ds(q * LANES, LANES)] = dst

    def grp_body(g, carry):
        for b in range(NBUF):
            i = g * NBUF + b
            compute_dst(i)
            pltpu.make_async_copy(table.at[ids_v.at[i]], bufs[b],
                                  gsems[b]).wait()
            pltpu.async_copy(bufs[b], acc_ref.at[dst_v.at[i]], ssems[b],
                             add=True)

        @pl.when(g < NGRP - 1)
        def _refill():
            for b in range(NBUF):
                i = (g + 1) * NBUF + b
                pltpu.make_async_copy(bufs[b], acc_ref.at[dst_v.at[i]],
                                      ssems[b]).wait()
                pltpu.async_copy(table.at[ids_v.at[i]], bufs[b], gsems[b])

        return carry

    lax.fori_loop(0, NGRP, grp_body, 0)

    for b in range(NBUF):
        i = NCH - NBUF + b
        pltpu.make_async_copy(bufs[b], acc_ref.at[dst_v.at[i]],
                              ssems[b]).wait()

    for t in range(4):
        pltpu.sync_copy(acc_ref.at[pl.ds(sbase + t * CHUNK, CHUNK)], obuf)
        pltpu.sync_copy(obuf, out.at[pl.ds(w * ROWS_W + t * CHUNK, CHUNK)])


@jax.jit
def _run(sidx, sw, vidx, vw, oidx, ow, table):
    mesh = plsc.VectorSubcoreMesh(core_axis_name="c", subcore_axis_name="s")

    def body(sidx, sw, vidx, vw, oidx, ow, table, out,
             ids_v, dst_v, acc,
             b0, b1, b2, b3, g0, g1, g2, g3, s0, s1, s2, s3):
        _body(sidx, sw, vidx, vw, oidx, ow, table, out,
              ids_v, dst_v, acc,
              (b0, b1, b2, b3), (g0, g1, g2, g3), (s0, s1, s2, s3))

    f = pl.kernel(
        body,
        out_type=jax.ShapeDtypeStruct((B, D), jnp.float32),
        mesh=mesh,
        scratch_types=[
            pltpu.VMEM((NCH, CHUNK), jnp.int32),
            pltpu.VMEM((NCH, CHUNK), jnp.int32),
            pltpu.VMEM_SHARED((NS * REGION, D), jnp.float32),
        ] + [pltpu.VMEM((CHUNK, D), jnp.float32) for _ in range(NBUF)]
          + [pltpu.SemaphoreType.DMA for _ in range(2 * NBUF)],
        compiler_params=pltpu.CompilerParams(use_tc_tiling_on_sc=False,
                                             needs_layout_passes=False),
    )
    return f(sidx, sw, vidx, vw, oidx, ow, table)


def kernel(subj_id, subj_w, verb_id, verb_w, obj_id, obj_w, table):
    shp = (NW, NCH_ARR, CHUNK)

    def wb(x):
        return lax.bitcast_convert_type(x, jnp.int32).reshape(shp)

    return _run(
        subj_id.astype(jnp.int32).reshape(shp), wb(subj_w),
        verb_id.astype(jnp.int32).reshape(shp), wb(verb_w),
        obj_id.astype(jnp.int32).reshape(shp), wb(obj_w),
        table)
